# Initial kernel scaffold; baseline (speedup 1.0000x reference)
import jax
import jax.numpy as jnp
from jax.experimental import pallas as pl


def kernel(x, edge_index, edge_attr, W_f, b_f, W_s, b_s):
    raise NotImplementedError("write your pallas kernel here")


# speedup vs baseline: 3.2301x; 3.2301x over previous
import functools

import jax
import jax.numpy as jnp
from jax import lax
from jax.experimental import pallas as pl
from jax.experimental.pallas import tpu as pltpu
from jax.experimental.pallas import tpu_sc as plsc

N = 10000
E = 320000
C = 64
ED = 128
Z = 2 * C + ED

NC = 2
NS = 16
NW = NC * NS
EPT = E // NW
CH = 80
K = 5
WAVES = EPT // (K * CH)
RPT = N // NS

def _sc_gather_body(x_hbm, dst_hbm, src_hbm, xi_hbm, xj_hbm,
                    dst_v, src_v, bi, bj, sem_g, sem_w):
    wid = lax.axis_index("s") * NC + lax.axis_index("c")
    base = wid * EPT
    pltpu.sync_copy(dst_hbm.at[pl.ds(base, EPT)], dst_v)
    pltpu.sync_copy(src_hbm.at[pl.ds(base, EPT)], src_v)

    def wave(w, carry):
        off = w * (K * CH)
        gathers = []
        for b in range(K):
            o = off + b * CH
            gathers.append(pltpu.async_copy(
                x_hbm.at[dst_v.at[pl.ds(o, CH)]], bi.at[b], sem_g))
            gathers.append(pltpu.async_copy(
                x_hbm.at[src_v.at[pl.ds(o, CH)]], bj.at[b], sem_g))
        for cp in gathers:
            cp.wait()
        writes = []
        for b in range(K):
            o = base + off + b * CH
            writes.append(pltpu.async_copy(bi.at[b], xi_hbm.at[pl.ds(o, CH)], sem_w))
            writes.append(pltpu.async_copy(bj.at[b], xj_hbm.at[pl.ds(o, CH)], sem_w))
        for cp in writes:
            cp.wait()
        return carry

    lax.fori_loop(0, WAVES, wave, 0)


def _sc_scatter_body(msg_hbm, dst_hbm, zero_hbm, out_hbm,
                     mbuf, ibuf, acc, sem_l, sem_s):
    cid = lax.axis_index("c")
    sid = lax.axis_index("s")
    wid = sid * NC + cid
    base = wid * EPT
    rows = sid * RPT
    pltpu.sync_copy(zero_hbm.at[pl.ds(rows, RPT)], acc.at[pl.ds(rows, RPT)])
    plsc.subcore_barrier()

    def wave(w, carry):
        off = base + w * (K * CH)
        loads = []
        for b in range(K):
            o = off + b * CH
            loads.append(pltpu.async_copy(
                dst_hbm.at[pl.ds(o, CH)], ibuf.at[b], sem_l))
            loads.append(pltpu.async_copy(
                msg_hbm.at[pl.ds(o, CH)], mbuf.at[b], sem_l))
        for cp in loads:
            cp.wait()
        adds = []
        for b in range(K):
            adds.append(pltpu.async_copy(
                mbuf.at[b], acc.at[ibuf.at[b]], sem_s, add=True))
        for cp in adds:
            cp.wait()
        return carry

    lax.fori_loop(0, WAVES, wave, 0)
    plsc.subcore_barrier()
    pltpu.sync_copy(acc.at[pl.ds(rows, RPT)], out_hbm.at[cid, pl.ds(rows, RPT)])


BE = 2560


def _dense_body(xi_ref, xj_ref, ea_ref, w_ref, b_ref, out_ref):
    z = jnp.concatenate([xi_ref[...], xj_ref[...], ea_ref[...]], axis=-1)
    gs = jnp.dot(z, w_ref[...], preferred_element_type=jnp.float32) + b_ref[...]
    g = gs[:, :C]
    s = gs[:, C:]
    gate = 1.0 / (1.0 + jnp.exp(-g))
    core = jnp.maximum(s, 0.0) + jnp.log1p(jnp.exp(-jnp.abs(s)))
    out_ref[...] = gate * core


def _dense(xi, xj, edge_attr, w_cat, b_cat):
    return pl.pallas_call(
        _dense_body,
        grid=(E // BE,),
        in_specs=[
            pl.BlockSpec((BE, C), lambda i: (i, 0)),
            pl.BlockSpec((BE, C), lambda i: (i, 0)),
            pl.BlockSpec((BE, ED), lambda i: (i, 0)),
            pl.BlockSpec((Z, 2 * C), lambda i: (0, 0)),
            pl.BlockSpec((1, 2 * C), lambda i: (0, 0)),
        ],
        out_specs=pl.BlockSpec((BE, C), lambda i: (i, 0)),
        out_shape=jax.ShapeDtypeStruct((E, C), jnp.float32),
    )(xi, xj, edge_attr, w_cat, b_cat)


BN = 2000


def _combine_body(x_ref, p_ref, out_ref):
    out_ref[...] = jnp.maximum(x_ref[...] + p_ref[0] + p_ref[1], 0.0)


def _combine(x, partials):
    return pl.pallas_call(
        _combine_body,
        grid=(N // BN,),
        in_specs=[
            pl.BlockSpec((BN, C), lambda i: (i, 0)),
            pl.BlockSpec((NC, BN, C), lambda i: (0, i, 0)),
        ],
        out_specs=pl.BlockSpec((BN, C), lambda i: (i, 0)),
        out_shape=jax.ShapeDtypeStruct((N, C), jnp.float32),
    )(x, partials)


@functools.cache
def _sc_kernels():
    mesh = plsc.VectorSubcoreMesh(core_axis_name="c", subcore_axis_name="s",
                                  num_cores=NC, num_subcores=NS)
    params = pltpu.CompilerParams(use_tc_tiling_on_sc=False)
    gather = pl.kernel(
        _sc_gather_body,
        compiler_params=params,
        out_type=(jax.ShapeDtypeStruct((E, C), jnp.float32),
                  jax.ShapeDtypeStruct((E, C), jnp.float32)),
        mesh=mesh,
        scratch_types=[
            pltpu.VMEM((EPT,), jnp.int32),
            pltpu.VMEM((EPT,), jnp.int32),
            pltpu.VMEM((K, CH, C), jnp.float32),
            pltpu.VMEM((K, CH, C), jnp.float32),
            pltpu.SemaphoreType.DMA,
            pltpu.SemaphoreType.DMA,
        ],
    )
    scatter = pl.kernel(
        _sc_scatter_body,
        compiler_params=params,
        out_type=jax.ShapeDtypeStruct((NC, N, C), jnp.float32),
        mesh=mesh,
        scratch_types=[
            pltpu.VMEM((K, CH, C), jnp.float32),
            pltpu.VMEM((K, CH), jnp.int32),
            pltpu.VMEM_SHARED((N, C), jnp.float32),
            pltpu.SemaphoreType.DMA,
            pltpu.SemaphoreType.DMA,
        ],
    )
    return gather, scatter


def kernel(x, edge_index, edge_attr, W_f, b_f, W_s, b_s):
    sc_gather, sc_scatter = _sc_kernels()
    src = edge_index[0].astype(jnp.int32)
    dst = edge_index[1].astype(jnp.int32)
    xi, xj = sc_gather(x, dst, src)
    w_cat = jnp.concatenate([W_f, W_s], axis=1)
    b_cat = jnp.concatenate([b_f, b_s]).reshape(1, 2 * C)
    msg = _dense(xi, xj, edge_attr, w_cat, b_cat)
    partials = sc_scatter(msg, dst, jnp.zeros((N, C), jnp.float32))
    return _combine(x, partials)


# speedup vs baseline: 6.8078x; 2.1076x over previous
import functools

import jax
import jax.numpy as jnp
from jax import lax
from jax.experimental import pallas as pl
from jax.experimental.pallas import tpu as pltpu
from jax.experimental.pallas import tpu_sc as plsc

N = 10000
E = 320000
EH = E // 2
C = 64
ED = 128
Z = 2 * C + ED

NC = 2
NS = 16
NW = NC * NS
RPT = N // NS

G_CH = 40
G_K = 5
G_WV = G_K * G_CH
G_R = 2
G_EPT = E // NW
G_NWAVE = G_EPT // G_WV

S_CH = 40
S_K = 5
S_WV = S_K * S_CH
S_RPT = EH // NW
S_NWAVE = S_RPT // S_WV


def _sc_gather_body(x_hbm, dst_hbm, src_hbm, xij_hbm,
                    tbl, dst_v, src_v, bi, bj, sem_g, sem_w):
    cid = lax.axis_index("c")
    sid = lax.axis_index("s")
    wid = sid * NC + cid
    base = wid * G_EPT
    rows = sid * RPT
    pltpu.sync_copy(x_hbm.at[pl.ds(rows, RPT)], tbl.at[pl.ds(rows, RPT)])
    pltpu.sync_copy(dst_hbm.at[pl.ds(base, G_EPT)], dst_v)
    pltpu.sync_copy(src_hbm.at[pl.ds(base, G_EPT)], src_v)
    plsc.subcore_barrier()

    def wave(w, carry):
        s = w % G_R

        @pl.when(w >= G_R)
        def _drain():
            pltpu.make_async_copy(
                bi.at[s], xij_hbm.at[pl.ds(base, G_WV), pl.ds(0, C)], sem_w).wait()
            pltpu.make_async_copy(
                bj.at[s], xij_hbm.at[pl.ds(base, G_WV), pl.ds(C, C)], sem_w).wait()

        cps = []
        for b in range(G_K):
            o = w * G_WV + b * G_CH
            cps.append(pltpu.async_copy(
                tbl.at[dst_v.at[pl.ds(o, G_CH)]],
                bi.at[s, pl.ds(b * G_CH, G_CH)], sem_g))
            cps.append(pltpu.async_copy(
                tbl.at[src_v.at[pl.ds(o, G_CH)]],
                bj.at[s, pl.ds(b * G_CH, G_CH)], sem_g))
        for cp in cps:
            cp.wait()
        o = base + w * G_WV
        pltpu.async_copy(bi.at[s], xij_hbm.at[pl.ds(o, G_WV), pl.ds(0, C)], sem_w)
        pltpu.async_copy(bj.at[s], xij_hbm.at[pl.ds(o, G_WV), pl.ds(C, C)], sem_w)
        return carry

    lax.fori_loop(0, G_NWAVE, wave, 0)
    for _ in range(G_R):
        pltpu.make_async_copy(
            bi.at[0], xij_hbm.at[pl.ds(base, G_WV), pl.ds(0, C)], sem_w).wait()
        pltpu.make_async_copy(
            bj.at[0], xij_hbm.at[pl.ds(base, G_WV), pl.ds(C, C)], sem_w).wait()


def _sc_scatter_body(msg_hbm, dst_hbm, zero_hbm, out_hbm,
                     mba, mbb, iba, ibb, acc, sem_l, sem_s):
    cid = lax.axis_index("c")
    sid = lax.axis_index("s")
    wid = sid * NC + cid
    base = wid * S_RPT
    rows = sid * RPT
    pltpu.sync_copy(zero_hbm.at[pl.ds(rows, RPT)], acc.at[pl.ds(rows, RPT)])
    plsc.subcore_barrier()

    def wave(w, carry):
        loads = []
        for b in range(S_K):
            o = base + w * S_WV + b * S_CH
            loads.append(pltpu.async_copy(
                dst_hbm.at[pl.ds(o, S_CH)], iba.at[b], sem_l))
            loads.append(pltpu.async_copy(
                dst_hbm.at[pl.ds(EH + o, S_CH)], ibb.at[b], sem_l))
            loads.append(pltpu.async_copy(
                msg_hbm.at[pl.ds(o, S_CH), pl.ds(0, C)], mba.at[b], sem_l))
            loads.append(pltpu.async_copy(
                msg_hbm.at[pl.ds(o, S_CH), pl.ds(C, C)], mbb.at[b], sem_l))
        for cp in loads:
            cp.wait()
        adds = []
        for b in range(S_K):
            adds.append(pltpu.async_copy(
                mba.at[b], acc.at[iba.at[b]], sem_s, add=True))
            adds.append(pltpu.async_copy(
                mbb.at[b], acc.at[ibb.at[b]], sem_s, add=True))
        for cp in adds:
            cp.wait()
        return carry

    lax.fori_loop(0, S_NWAVE, wave, 0)
    plsc.subcore_barrier()
    pltpu.sync_copy(acc.at[pl.ds(rows, RPT)], out_hbm.at[cid, pl.ds(rows, RPT)])


@functools.cache
def _sc_kernels():
    mesh = plsc.VectorSubcoreMesh(core_axis_name="c", subcore_axis_name="s",
                                  num_cores=NC, num_subcores=NS)
    params = pltpu.CompilerParams(use_tc_tiling_on_sc=False)
    gather = pl.kernel(
        _sc_gather_body,
        out_type=jax.ShapeDtypeStruct((E, 2 * C), jnp.float32),
        mesh=mesh,
        compiler_params=params,
        scratch_types=[
            pltpu.VMEM_SHARED((N, C), jnp.float32),
            pltpu.VMEM((G_EPT,), jnp.int32),
            pltpu.VMEM((G_EPT,), jnp.int32),
            pltpu.VMEM((G_R, G_WV, C), jnp.float32),
            pltpu.VMEM((G_R, G_WV, C), jnp.float32),
            pltpu.SemaphoreType.DMA,
            pltpu.SemaphoreType.DMA,
        ],
    )
    scatter = pl.kernel(
        _sc_scatter_body,
        out_type=jax.ShapeDtypeStruct((NC, N, C), jnp.float32),
        mesh=mesh,
        compiler_params=params,
        scratch_types=[
            pltpu.VMEM((S_K, S_CH, C), jnp.float32),
            pltpu.VMEM((S_K, S_CH, C), jnp.float32),
            pltpu.VMEM((S_K, S_CH), jnp.int32),
            pltpu.VMEM((S_K, S_CH), jnp.int32),
            pltpu.VMEM_SHARED((N, C), jnp.float32),
            pltpu.SemaphoreType.DMA,
            pltpu.SemaphoreType.DMA,
        ],
    )
    return gather, scatter


BH = 1280
NBLK = EH // BH


def _dense_body(xa_ref, xb_ref, ea_ref, eb_ref, w_ref, b_ref, out_ref):
    za = jnp.concatenate([xa_ref[...], ea_ref[...]], axis=-1)
    zb = jnp.concatenate([xb_ref[...], eb_ref[...]], axis=-1)
    ga = jnp.dot(za, w_ref[...], preferred_element_type=jnp.float32) + b_ref[...]
    gb = jnp.dot(zb, w_ref[...], preferred_element_type=jnp.float32) + b_ref[...]

    def act(gs):
        g = gs[:, :C]
        s = gs[:, C:]
        gate = 1.0 / (1.0 + jnp.exp(-g))
        core = jnp.maximum(s, 0.0) + jnp.log1p(jnp.exp(-jnp.abs(s)))
        return gate * core

    out_ref[...] = jnp.concatenate([act(ga), act(gb)], axis=-1)


def _dense(xij, edge_attr, w_cat, b_cat):
    return pl.pallas_call(
        _dense_body,
        grid=(NBLK,),
        in_specs=[
            pl.BlockSpec((BH, 2 * C), lambda i: (i, 0)),
            pl.BlockSpec((BH, 2 * C), lambda i: (i + NBLK, 0)),
            pl.BlockSpec((BH, ED), lambda i: (i, 0)),
            pl.BlockSpec((BH, ED), lambda i: (i + NBLK, 0)),
            pl.BlockSpec((Z, 2 * C), lambda i: (0, 0)),
            pl.BlockSpec((1, 2 * C), lambda i: (0, 0)),
        ],
        out_specs=pl.BlockSpec((BH, 2 * C), lambda i: (i, 0)),
        out_shape=jax.ShapeDtypeStruct((EH, 2 * C), jnp.float32),
    )(xij, xij, edge_attr, edge_attr, w_cat, b_cat)


BN = 2000


def _combine_body(x_ref, p_ref, out_ref):
    out_ref[...] = jnp.maximum(x_ref[...] + p_ref[0] + p_ref[1], 0.0)


def _combine(x, partials):
    return pl.pallas_call(
        _combine_body,
        grid=(N // BN,),
        in_specs=[
            pl.BlockSpec((BN, C), lambda i: (i, 0)),
            pl.BlockSpec((NC, BN, C), lambda i: (0, i, 0)),
        ],
        out_specs=pl.BlockSpec((BN, C), lambda i: (i, 0)),
        out_shape=jax.ShapeDtypeStruct((N, C), jnp.float32),
    )(x, partials)


def kernel(x, edge_index, edge_attr, W_f, b_f, W_s, b_s):
    sc_gather, sc_scatter = _sc_kernels()
    src = edge_index[0].astype(jnp.int32)
    dst = edge_index[1].astype(jnp.int32)
    xij = sc_gather(x, dst, src)
    w_cat = jnp.concatenate([W_f, W_s], axis=1)
    b_cat = jnp.concatenate([b_f, b_s]).reshape(1, 2 * C)
    msg2 = _dense(xij, edge_attr, w_cat, b_cat)
    partials = sc_scatter(msg2, dst, jnp.zeros((N, C), jnp.float32))
    return _combine(x, partials)


# speedup vs baseline: 8.6981x; 1.2777x over previous
import functools

import jax
import jax.numpy as jnp
from jax import lax
from jax.experimental import pallas as pl
from jax.experimental.pallas import tpu as pltpu
from jax.experimental.pallas import tpu_sc as plsc

N = 10000
E = 320000
C = 64
ED = 128
Z = 2 * C + ED

NC = 2
NS = 16
NW = NC * NS
RPT = N // NS

CHAINS = ((0, 192000), (192000, 128000))

G_CH = 40
G_K = 5
G_WV = G_K * G_CH
G_R = 2

S_CH = 40
S_K = 5
S_WV = S_K * S_CH


def _sc_gather_body(ept, nwave, e_off, x_hbm, ei_hbm, xij_hbm,
                    tbl, dst_v, src_v, bi, bj, sem_g, sem_w):
    cid = lax.axis_index("c")
    sid = lax.axis_index("s")
    wid = sid * NC + cid
    base = wid * ept
    rows = sid * RPT
    pltpu.sync_copy(x_hbm.at[pl.ds(rows, RPT)], tbl.at[pl.ds(rows, RPT)])
    pltpu.sync_copy(ei_hbm.at[1, pl.ds(e_off + base, ept)], dst_v)
    pltpu.sync_copy(ei_hbm.at[0, pl.ds(e_off + base, ept)], src_v)
    plsc.subcore_barrier()

    def wave(w, carry):
        s = w % G_R

        @pl.when(w >= G_R)
        def _drain():
            pltpu.make_async_copy(
                bi.at[s], xij_hbm.at[pl.ds(base, G_WV), pl.ds(0, C)], sem_w).wait()
            pltpu.make_async_copy(
                bj.at[s], xij_hbm.at[pl.ds(base, G_WV), pl.ds(C, C)], sem_w).wait()

        cps = []
        for b in range(G_K):
            o = w * G_WV + b * G_CH
            cps.append(pltpu.async_copy(
                tbl.at[dst_v.at[pl.ds(o, G_CH)]],
                bi.at[s, pl.ds(b * G_CH, G_CH)], sem_g))
            cps.append(pltpu.async_copy(
                tbl.at[src_v.at[pl.ds(o, G_CH)]],
                bj.at[s, pl.ds(b * G_CH, G_CH)], sem_g))
        for cp in cps:
            cp.wait()
        o = base + w * G_WV
        pltpu.async_copy(bi.at[s], xij_hbm.at[pl.ds(o, G_WV), pl.ds(0, C)], sem_w)
        pltpu.async_copy(bj.at[s], xij_hbm.at[pl.ds(o, G_WV), pl.ds(C, C)], sem_w)
        return carry

    lax.fori_loop(0, nwave, wave, 0)
    for _ in range(G_R):
        pltpu.make_async_copy(
            bi.at[0], xij_hbm.at[pl.ds(base, G_WV), pl.ds(0, C)], sem_w).wait()
        pltpu.make_async_copy(
            bj.at[0], xij_hbm.at[pl.ds(base, G_WV), pl.ds(C, C)], sem_w).wait()


def _sc_scatter_body(rpt, nwave, e_off, eh, msg_hbm, ei_hbm, zero_hbm, out_hbm,
                     mba, mbb, iba, ibb, acc, sem_l, sem_s):
    cid = lax.axis_index("c")
    sid = lax.axis_index("s")
    wid = sid * NC + cid
    base = wid * rpt
    rows = sid * RPT
    pltpu.sync_copy(zero_hbm.at[pl.ds(rows, RPT)], acc.at[pl.ds(rows, RPT)])
    plsc.subcore_barrier()

    def wave(w, carry):
        loads = []
        for b in range(S_K):
            o = base + w * S_WV + b * S_CH
            loads.append(pltpu.async_copy(
                ei_hbm.at[1, pl.ds(e_off + o, S_CH)], iba.at[b], sem_l))
            loads.append(pltpu.async_copy(
                ei_hbm.at[1, pl.ds(e_off + eh + o, S_CH)], ibb.at[b], sem_l))
            loads.append(pltpu.async_copy(
                msg_hbm.at[pl.ds(o, S_CH), pl.ds(0, C)], mba.at[b], sem_l))
            loads.append(pltpu.async_copy(
                msg_hbm.at[pl.ds(o, S_CH), pl.ds(C, C)], mbb.at[b], sem_l))
        for cp in loads:
            cp.wait()
        adds = []
        for b in range(S_K):
            adds.append(pltpu.async_copy(
                mba.at[b], acc.at[iba.at[b]], sem_s, add=True))
            adds.append(pltpu.async_copy(
                mbb.at[b], acc.at[ibb.at[b]], sem_s, add=True))
        for cp in adds:
            cp.wait()
        return carry

    lax.fori_loop(0, nwave, wave, 0)
    plsc.subcore_barrier()
    pltpu.sync_copy(acc.at[pl.ds(rows, RPT)],
                    out_hbm.at[pl.ds(rows, RPT), pl.ds(cid * C, C)])


@functools.cache
def _sc_kernels():
    mesh = plsc.VectorSubcoreMesh(core_axis_name="c", subcore_axis_name="s",
                                  num_cores=NC, num_subcores=NS)
    params = pltpu.CompilerParams(use_tc_tiling_on_sc=False)
    gathers, scatters = [], []
    for e_off, ec in CHAINS:
        ept = ec // NW
        gathers.append(pl.kernel(
            functools.partial(_sc_gather_body, ept, ept // G_WV, e_off),
            out_type=jax.ShapeDtypeStruct((ec, 2 * C), jnp.float32),
            mesh=mesh,
            compiler_params=params,
            scratch_types=[
                pltpu.VMEM_SHARED((N, C), jnp.float32),
                pltpu.VMEM((ept,), jnp.int32),
                pltpu.VMEM((ept,), jnp.int32),
                pltpu.VMEM((G_R, G_WV, C), jnp.float32),
                pltpu.VMEM((G_R, G_WV, C), jnp.float32),
                pltpu.SemaphoreType.DMA,
                pltpu.SemaphoreType.DMA,
            ],
        ))
        rpt = (ec // 2) // NW
        scatters.append(pl.kernel(
            functools.partial(_sc_scatter_body, rpt, rpt // S_WV, e_off, ec // 2),
            out_type=jax.ShapeDtypeStruct((N, 2 * C), jnp.float32),
            mesh=mesh,
            compiler_params=params,
            scratch_types=[
                pltpu.VMEM((S_K, S_CH, C), jnp.float32),
                pltpu.VMEM((S_K, S_CH, C), jnp.float32),
                pltpu.VMEM((S_K, S_CH), jnp.int32),
                pltpu.VMEM((S_K, S_CH), jnp.int32),
                pltpu.VMEM_SHARED((N, C), jnp.float32),
                pltpu.SemaphoreType.DMA,
                pltpu.SemaphoreType.DMA,
            ],
        ))
    return gathers, scatters


BH = 2000


def _dense_body(xa_ref, xb_ref, ea_ref, eb_ref, w_ref, b_ref, out_ref):
    za = jnp.concatenate([xa_ref[...], ea_ref[...]], axis=-1)
    zb = jnp.concatenate([xb_ref[...], eb_ref[...]], axis=-1)
    ga = jnp.dot(za, w_ref[...], preferred_element_type=jnp.float32) + b_ref[...]
    gb = jnp.dot(zb, w_ref[...], preferred_element_type=jnp.float32) + b_ref[...]

    def act(gs):
        g = gs[:, :C]
        s = gs[:, C:]
        gate = 1.0 / (1.0 + jnp.exp(-g))
        core = jnp.maximum(s, 0.0) + jnp.log1p(jnp.exp(-jnp.abs(s)))
        return gate * core

    out_ref[...] = jnp.concatenate([act(ga), act(gb)], axis=-1)


def _dense(xij, edge_attr, w_cat, b_cat, e_off, ec):
    eh = ec // 2
    nblk = eh // BH
    ea_a = e_off // BH
    ea_b = (e_off + eh) // BH
    return pl.pallas_call(
        _dense_body,
        grid=(nblk,),
        in_specs=[
            pl.BlockSpec((BH, 2 * C), lambda i: (i, 0)),
            pl.BlockSpec((BH, 2 * C), lambda i, n=nblk: (i + n, 0)),
            pl.BlockSpec((BH, ED), lambda i, o=ea_a: (i + o, 0)),
            pl.BlockSpec((BH, ED), lambda i, o=ea_b: (i + o, 0)),
            pl.BlockSpec((Z, 2 * C), lambda i: (0, 0)),
            pl.BlockSpec((1, 2 * C), lambda i: (0, 0)),
        ],
        out_specs=pl.BlockSpec((BH, 2 * C), lambda i: (i, 0)),
        out_shape=jax.ShapeDtypeStruct((eh, 2 * C), jnp.float32),
    )(xij, xij, edge_attr, edge_attr, w_cat, b_cat)


BN = 2000


def _combine_body(x_ref, pa_ref, pb_ref, out_ref):
    s = (x_ref[...] + pa_ref[:, :C] + pa_ref[:, C:]
         + pb_ref[:, :C] + pb_ref[:, C:])
    out_ref[...] = jnp.maximum(s, 0.0)


def _combine(x, pa, pb):
    return pl.pallas_call(
        _combine_body,
        grid=(N // BN,),
        in_specs=[
            pl.BlockSpec((BN, C), lambda i: (i, 0)),
            pl.BlockSpec((BN, 2 * C), lambda i: (i, 0)),
            pl.BlockSpec((BN, 2 * C), lambda i: (i, 0)),
        ],
        out_specs=pl.BlockSpec((BN, C), lambda i: (i, 0)),
        out_shape=jax.ShapeDtypeStruct((N, C), jnp.float32),
    )(x, pa, pb)


def kernel(x, edge_index, edge_attr, W_f, b_f, W_s, b_s):
    gathers, scatters = _sc_kernels()
    ei = edge_index if edge_index.dtype == jnp.int32 else edge_index.astype(jnp.int32)
    w_cat = jnp.concatenate([W_f, W_s], axis=1)
    b_cat = jnp.concatenate([b_f, b_s]).reshape(1, 2 * C)
    zeros = jnp.zeros((N, C), jnp.float32)
    partials = []
    for (e_off, ec), g, s in zip(CHAINS, gathers, scatters):
        xij = g(x, ei)
        msg2 = _dense(xij, edge_attr, w_cat, b_cat, e_off, ec)
        partials.append(s(msg2, ei, zeros))
    return _combine(x, partials[0], partials[1])
